# baseline (speedup 1.0000x reference)
import jax
import jax.numpy as jnp
from jax import lax
from jax.experimental import pallas as pl
from jax.experimental.pallas import tpu as pltpu


def _tree_select(cands, bits):
    vals = list(cands)
    for b in reversed(bits):
        vals = [jnp.where(b, vals[2 * i + 1], vals[2 * i])
                for i in range(len(vals) // 2)]
    return vals[0]


def _vq_kernel(x_ref, c_ref, o_ref, srt_ref, mid_ref):
    k, lanes = c_ref.shape

    @pl.when(pl.program_id(0) == 0)
    def _prep():
        c = c_ref[...]
        ci = c[:, None, :]
        cj = c[None, :, :]
        ii = lax.broadcasted_iota(jnp.int32, (k, k, 1), 0)
        jj = lax.broadcasted_iota(jnp.int32, (k, k, 1), 1)
        rank = jnp.sum(
            jnp.where((cj < ci) | ((cj == ci) & (jj < ii)), 1, 0),
            axis=1,
        )
        rr = lax.broadcasted_iota(jnp.int32, (k, k, 1), 0)
        oh = (rank[None, :, :] == rr).astype(c.dtype)
        srt = jnp.sum(oh * c[None, :, :], axis=1)
        nxt = jnp.concatenate([srt[1:], srt[k - 1:]], axis=0)
        srt_ref[...] = srt
        mid_ref[...] = 0.5 * (srt + nxt)

    def m(j):
        return mid_ref[j:j + 1, :]

    x = x_ref[...].reshape(-1, c_ref.shape[1])
    levels = k.bit_length() - 1
    bits = []
    for l in range(levels):
        step = 1 << (levels - 1 - l)
        cands = [m(p * 2 * step + step - 1) for p in range(1 << l)]
        boundary = _tree_select(cands, bits)
        bits.append(x > boundary)
    vals = [srt_ref[j:j + 1, :] for j in range(k)]
    o_ref[...] = _tree_select(vals, bits).reshape(o_ref.shape)


def kernel(melspecs, centroids):
    b, t, n_mels = melspecs.shape
    k = centroids.shape[1]
    ct = centroids.T
    bb = 2
    grid = (b // bb,)
    out = pl.pallas_call(
        _vq_kernel,
        grid=grid,
        in_specs=[
            pl.BlockSpec((bb, t, n_mels), lambda i: (i, 0, 0)),
            pl.BlockSpec((k, n_mels), lambda i: (0, 0)),
        ],
        out_specs=pl.BlockSpec((bb, t, n_mels), lambda i: (i, 0, 0)),
        out_shape=jax.ShapeDtypeStruct((b, t, n_mels), melspecs.dtype),
        scratch_shapes=[
            pltpu.VMEM((k, n_mels), melspecs.dtype),
            pltpu.VMEM((k, n_mels), melspecs.dtype),
        ],
    )(melspecs, ct)
    return out
